# baseline (speedup 1.0000x reference)
import jax
import jax.numpy as jnp
from jax.experimental import pallas as pl
from jax.experimental.pallas import tpu as pltpu

N_ROWS = 64
C = 256
_HW = (56 * 56, 28 * 28, 14 * 14, 7 * 7, 4 * 4)
_CHUNKS = (8, 4, 2, 1, 1)


def _body(lids, chs, i0, i1, i2, i3, i4, o0, o1, o2, o3, o4,
          f0, f1, f2, f3, f4, sem, fsem):
    ins = (i0, i1, i2, i3, i4)
    outs = (o0, o1, o2, o3, o4)
    fills = (f0, f1, f2, f3, f4)

    descs = []
    for li, hw in enumerate(_HW):
        rows = N_ROWS * C
        nch = _CHUNKS[li]
        rpc = rows // nch
        for k in range(nch):
            d = pltpu.make_async_copy(
                ins[li].at[pl.ds(k * rpc, rpc)],
                outs[li].at[pl.ds(k * rpc, rpc)],
                sem,
            )
            d.start()
            descs.append(d)
    for li, hw in enumerate(_HW):
        fills[li][...] = jnp.full((hw,), -1000.0, jnp.float32)
    for d in descs:
        d.wait()

    started = []
    for i in range(N_ROWS):
        lid = lids[i]
        c = chs[i]
        for li in range(5):
            @pl.when(lid == li)
            def _(li=li, i=i, c=c):
                pltpu.make_async_copy(
                    fills[li], outs[li].at[i * C + c], fsem
                ).start()
    for i in range(N_ROWS):
        lid = lids[i]
        c = chs[i]
        for li in range(5):
            @pl.when(lid == li)
            def _(li=li, i=i, c=c):
                pltpu.make_async_copy(
                    fills[li], outs[li].at[i * C + c], fsem
                ).wait()


def kernel(act_0, act_1, act_2, act_3, act_pool, indices, x):
    del x
    acts = (act_0, act_1, act_2, act_3, act_pool)
    layer_ids = (indices // C).astype(jnp.int32)
    ch = (indices % C).astype(jnp.int32)
    flat = [a.reshape(N_ROWS * C, hw) for a, hw in zip(acts, _HW)]

    any_spec = pl.BlockSpec(memory_space=pl.ANY)
    smem_spec = pl.BlockSpec(memory_space=pltpu.SMEM)

    outs = pl.pallas_call(
        _body,
        in_specs=[smem_spec, smem_spec] + [any_spec] * 5,
        out_specs=[any_spec] * 5,
        out_shape=[jax.ShapeDtypeStruct(f.shape, f.dtype) for f in flat],
        scratch_shapes=[pltpu.VMEM((hw,), jnp.float32) for hw in _HW]
        + [pltpu.SemaphoreType.DMA, pltpu.SemaphoreType.DMA],
    )(layer_ids, ch, *flat)

    return tuple(o.reshape(a.shape) for o, a in zip(outs, acts))


# speedup vs baseline: 8.3988x; 8.3988x over previous
import jax
import jax.numpy as jnp
from jax.experimental import pallas as pl
from jax.experimental.pallas import tpu as pltpu

N_ROWS = 64
C = 256
_HW = (56 * 56, 28 * 28, 14 * 14, 7 * 7, 4 * 4)
_BLOCK = (None, (8, 256), (32, 256), (64, 256), (64, 256))


def _scatter0_body(lids, chs, iflat, oflat, fill, sem):
    del iflat
    fill[...] = jnp.full((_HW[0],), -1000.0, jnp.float32)
    for i in range(N_ROWS):
        lid = lids[i]
        c = chs[i]

        @pl.when(lid == 0)
        def _(i=i, c=c):
            pltpu.make_async_copy(fill, oflat.at[i * C + c], sem).start()
    for i in range(N_ROWS):
        lid = lids[i]
        c = chs[i]

        @pl.when(lid == 0)
        def _(i=i, c=c):
            pltpu.make_async_copy(fill, oflat.at[i * C + c], sem).wait()


def _ablate0(a, layer_ids, ch):
    flat = a.reshape(N_ROWS * C, _HW[0])
    out = pl.pallas_call(
        _scatter0_body,
        in_specs=[pl.BlockSpec(memory_space=pltpu.SMEM),
                  pl.BlockSpec(memory_space=pltpu.SMEM),
                  pl.BlockSpec(memory_space=pl.ANY)],
        out_specs=pl.BlockSpec(memory_space=pl.ANY),
        out_shape=jax.ShapeDtypeStruct(flat.shape, flat.dtype),
        input_output_aliases={2: 0},
        scratch_shapes=[pltpu.VMEM((_HW[0],), jnp.float32),
                        pltpu.SemaphoreType.DMA],
    )(layer_ids, ch, flat)
    return out.reshape(a.shape)


def _make_body(li, rpb, cpb, hw):
    def body(lids, chs, ain, aout):
        b0 = pl.program_id(0)
        b1 = pl.program_id(1)
        aout[...] = ain[...]
        for r in range(rpb):
            i = b0 * rpb + r
            lid = lids[i]
            c = chs[i]

            @pl.when((lid == li) & (c // cpb == b1))
            def _(r=r, c=c):
                aout[r, c % cpb, :] = jnp.full((hw,), -1000.0, jnp.float32)

    return body


def _ablate_level(li, a, layer_ids, ch):
    hw = _HW[li]
    rpb, cpb = _BLOCK[li]
    flat = a.reshape(N_ROWS, C, hw)
    spec = pl.BlockSpec((rpb, cpb, hw), lambda b0, b1, lids, chs: (b0, b1, 0))
    grid_spec = pltpu.PrefetchScalarGridSpec(
        num_scalar_prefetch=2,
        grid=(N_ROWS // rpb, C // cpb),
        in_specs=[spec],
        out_specs=spec,
    )
    out = pl.pallas_call(
        _make_body(li, rpb, cpb, hw),
        grid_spec=grid_spec,
        out_shape=jax.ShapeDtypeStruct(flat.shape, flat.dtype),
    )(layer_ids, ch, flat)
    return out.reshape(a.shape)


def kernel(act_0, act_1, act_2, act_3, act_pool, indices, x):
    del x
    acts = (act_0, act_1, act_2, act_3, act_pool)
    layer_ids = (indices // C).astype(jnp.int32)
    ch = (indices % C).astype(jnp.int32)
    out0 = _ablate0(act_0, layer_ids, ch)
    rest = tuple(
        _ablate_level(li, acts[li], layer_ids, ch) for li in range(1, 5)
    )
    return (out0,) + rest


# speedup vs baseline: 10.8156x; 1.2878x over previous
import jax
import jax.numpy as jnp
from jax.experimental import pallas as pl
from jax.experimental.pallas import tpu as pltpu

N_ROWS = 64
C = 256
_HW2D = ((56, 56), (28, 28), (14, 14), (7, 7), (4, 4))
_HW = tuple(h * w for h, w in _HW2D)
_BLOCK = (None, None, (32, 256), (64, 256), (64, 256))


def _make_scatter_body(li, h, w):
    def body(lids, chs, ain, aout, fill, sem):
        del ain
        fill[...] = jnp.full((h, w), -1000.0, jnp.float32)
        for i in range(N_ROWS):
            lid = lids[i]
            c = chs[i]

            @pl.when(lid == li)
            def _(i=i, c=c):
                pltpu.make_async_copy(fill, aout.at[i, c], sem).start()
        for i in range(N_ROWS):
            lid = lids[i]
            c = chs[i]

            @pl.when(lid == li)
            def _(i=i, c=c):
                pltpu.make_async_copy(fill, aout.at[i, c], sem).wait()

    return body


def _ablate_inplace(li, a, layer_ids, ch):
    h, w = _HW2D[li]
    return pl.pallas_call(
        _make_scatter_body(li, h, w),
        in_specs=[pl.BlockSpec(memory_space=pltpu.SMEM),
                  pl.BlockSpec(memory_space=pltpu.SMEM),
                  pl.BlockSpec(memory_space=pl.ANY)],
        out_specs=pl.BlockSpec(memory_space=pl.ANY),
        out_shape=jax.ShapeDtypeStruct(a.shape, a.dtype),
        input_output_aliases={2: 0},
        scratch_shapes=[pltpu.VMEM((h, w), jnp.float32),
                        pltpu.SemaphoreType.DMA],
    )(layer_ids, ch, a)


def _make_stream_body(li, rpb, cpb, hw):
    def body(lids, chs, ain, aout):
        b0 = pl.program_id(0)
        b1 = pl.program_id(1)
        aout[...] = ain[...]
        for r in range(rpb):
            i = b0 * rpb + r
            lid = lids[i]
            c = chs[i]

            @pl.when((lid == li) & (c // cpb == b1))
            def _(r=r, c=c):
                aout[r, c % cpb, :] = jnp.full((hw,), -1000.0, jnp.float32)

    return body


def _ablate_stream(li, a, layer_ids, ch):
    hw = _HW[li]
    rpb, cpb = _BLOCK[li]
    flat = a.reshape(N_ROWS, C, hw)
    spec = pl.BlockSpec((rpb, cpb, hw), lambda b0, b1, lids, chs: (b0, b1, 0))
    grid_spec = pltpu.PrefetchScalarGridSpec(
        num_scalar_prefetch=2,
        grid=(N_ROWS // rpb, C // cpb),
        in_specs=[spec],
        out_specs=spec,
    )
    out = pl.pallas_call(
        _make_stream_body(li, rpb, cpb, hw),
        grid_spec=grid_spec,
        out_shape=jax.ShapeDtypeStruct(flat.shape, flat.dtype),
    )(layer_ids, ch, flat)
    return out.reshape(a.shape)


def kernel(act_0, act_1, act_2, act_3, act_pool, indices, x):
    del x
    acts = (act_0, act_1, act_2, act_3, act_pool)
    layer_ids = (indices // C).astype(jnp.int32)
    ch = (indices % C).astype(jnp.int32)
    out0 = _ablate_inplace(0, act_0, layer_ids, ch)
    out1 = _ablate_inplace(1, act_1, layer_ids, ch)
    rest = tuple(
        _ablate_stream(li, acts[li], layer_ids, ch) for li in range(2, 5)
    )
    return (out0, out1) + rest
